# Initial kernel scaffold; baseline (speedup 1.0000x reference)
import jax
import jax.numpy as jnp
from jax.experimental import pallas as pl


def kernel(graph, node_feats, edge_feats, solv_graph, solv_node_feats, W_node, b_node, W_edge, b_edge, W_init, b_init, W_a, b_a, W_layers, b_layers, W_solv, b_solv, W_out, b_out):
    raise NotImplementedError("write your pallas kernel here")


# speedup vs baseline: 5.7285x; 5.7285x over previous
import functools

import jax
import jax.numpy as jnp
from jax import lax
from jax.experimental import pallas as pl
from jax.experimental.pallas import tpu as pltpu
from jax.experimental.pallas import tpu_sc as plsc

HIDDEN = 64
LAYERS = 4
N_NODES = 10000
N_EDGES = 320000
N_SNODES = 5000
N_SEDGES = 80000

PAIR_BLK = 3200
NODE_BLK = 2000
N_PAIRS = 160000

ROW = 128
K = 5
NC = 2
NS = 16
N_ROWS = N_EDGES // ROW
N_SROWS = N_SEDGES // ROW

_SC_PARAMS = pltpu.CompilerParams(use_tc_tiling_on_sc=False)


@functools.lru_cache(maxsize=None)
def _sc_mesh():
    return plsc.VectorSubcoreMesh(core_axis_name="c", subcore_axis_name="s")


def _leaky(x):
    return jnp.where(x > 0, x, 0.01 * x)


def _wrange(w, nw, count):
    per = count // nw
    rem = count % nw
    lo = w * per + jnp.minimum(w, rem)
    hi = lo + per + jnp.where(w < rem, 1, 0)
    return lo, hi


def _split_copy(src, dst, sid, rows):
    per = (rows // NS) // 8 * 8
    tail = rows - per * NS
    pltpu.sync_copy(src.at[pl.ds(sid * per, per)],
                    dst.at[pl.ds(sid * per, per)])
    if tail:
        @pl.when(sid == NS - 1)
        def _():
            pltpu.sync_copy(src.at[pl.ds(per * NS, tail)],
                            dst.at[pl.ds(per * NS, tail)])


def _split_copy3(src, dst3, cid, sid, rows):
    per = (rows // NS) // 8 * 8
    tail = rows - per * NS
    pltpu.sync_copy(src.at[pl.ds(sid * per, per)],
                    dst3.at[cid, pl.ds(sid * per, per)])
    if tail:
        @pl.when(sid == NS - 1)
        def _():
            pltpu.sync_copy(src.at[pl.ds(per * NS, tail)],
                            dst3.at[cid, pl.ds(per * NS, tail)])


def _scatter_add_phase(vals_hbm, idx_hbm, table, vbuf, ibuf, sem_v, sem_i,
                       lo, hi, base=0):
    def fetch(c, slot):
        pltpu.async_copy(vals_hbm.at[pl.ds((c - base) * (K * ROW), K * ROW)],
                         vbuf.at[slot], sem_v.at[slot])
        pltpu.async_copy(idx_hbm.at[c], ibuf.at[slot], sem_i.at[slot])

    def wait_fetch(slot):
        pltpu.make_async_copy(vals_hbm.at[pl.ds(0, K * ROW)],
                              vbuf.at[slot], sem_v.at[slot]).wait()
        pltpu.make_async_copy(idx_hbm.at[0], ibuf.at[slot],
                              sem_i.at[slot]).wait()

    def scat(slot):
        for j in range(K):
            pltpu.sync_copy(vbuf.at[slot, pl.ds(j * ROW, ROW)],
                            table.at[ibuf.at[slot, j]], add=True)

    @pl.when(lo < hi)
    def _():
        fetch(lo, 0)

    @pl.loop(lo, hi, step=2)
    def _(t):
        @pl.when(t + 1 < hi)
        def _():
            fetch(t + 1, 1)
        wait_fetch(0)
        scat(0)

        @pl.when(t + 2 < hi)
        def _():
            fetch(t + 2, 0)

        @pl.when(t + 1 < hi)
        def _():
            wait_fetch(1)
            scat(1)


def _gather_phase(table, gidx_hbm, gout_hbm, vbuf, ibuf, sem_i, sem_w,
                  lo, hi, base=0):
    def fetch(c, slot):
        pltpu.async_copy(gidx_hbm.at[c], ibuf.at[slot], sem_i.at[slot])

    def wait_fetch(slot):
        pltpu.make_async_copy(gidx_hbm.at[0], ibuf.at[slot],
                              sem_i.at[slot]).wait()

    def wait_write(slot):
        pltpu.make_async_copy(vbuf.at[slot],
                              gout_hbm.at[pl.ds(0, K * ROW)],
                              sem_w.at[slot]).wait()

    def proc(c, t, slot, first_use_cond):
        @pl.when(first_use_cond)
        def _():
            wait_write(slot)
        for j in range(K):
            pltpu.sync_copy(table.at[ibuf.at[slot, j]],
                            vbuf.at[slot, pl.ds(j * ROW, ROW)])
        pltpu.async_copy(vbuf.at[slot],
                         gout_hbm.at[pl.ds((c - base) * (K * ROW), K * ROW)],
                         sem_w.at[slot])

    @pl.when(lo < hi)
    def _():
        fetch(lo, 0)

    @pl.loop(lo, hi, step=2)
    def _(t):
        @pl.when(t + 1 < hi)
        def _():
            fetch(t + 1, 1)
        wait_fetch(0)
        proc(t, t, 0, t >= lo + 2)

        @pl.when(t + 2 < hi)
        def _():
            fetch(t + 2, 0)

        @pl.when(t + 1 < hi)
        def _():
            wait_fetch(1)
            proc(t + 1, t, 1, t >= lo + 1)

    @pl.when(lo < hi)
    def _():
        wait_write(0)

    @pl.when(lo + 1 < hi)
    def _():
        wait_write(1)


HC = (N_EDGES // ROW) // 2 // K


def _seg_chain(valsA, valsB, idx_hbm, tout_hbm, zeros_hbm,
               table, vbuf, ibuf, sem_v, sem_i, sid):
    _split_copy(zeros_hbm, table, sid, N_NODES)
    plsc.subcore_barrier()

    lo, hi = _wrange(sid, NS, 2 * HC)
    _scatter_add_phase(valsA, idx_hbm, table, vbuf, ibuf, sem_v, sem_i,
                       lo, jnp.minimum(hi, HC), base=0)
    _scatter_add_phase(valsB, idx_hbm, table, vbuf, ibuf, sem_v, sem_i,
                       jnp.maximum(lo, HC), hi, base=HC)
    plsc.subcore_barrier()

    _split_copy(table, tout_hbm, sid, N_NODES)


def _seg_body(dA_hbm, dB_hbm, bA_hbm, bB_hbm, dst2_hbm, src2_hbm, zeros_hbm,
              td_hbm, tb_hbm, table, vbuf, ibuf, sem_v, sem_i):
    cid = lax.axis_index("c")
    sid = lax.axis_index("s")

    @pl.when(cid == 0)
    def _():
        _seg_chain(dA_hbm, dB_hbm, dst2_hbm, td_hbm, zeros_hbm,
                   table, vbuf, ibuf, sem_v, sem_i, sid)

    @pl.when(cid == 1)
    def _():
        _seg_chain(bA_hbm, bB_hbm, src2_hbm, tb_hbm, zeros_hbm,
                   table, vbuf, ibuf, sem_v, sem_i, sid)


@jax.jit
def _sparse_seg(dA, dB, bA, bB, dst2, src2, zeros):
    f = pl.kernel(
        _seg_body,
        out_type=[
            jax.ShapeDtypeStruct((N_NODES, HIDDEN), jnp.float32),
            jax.ShapeDtypeStruct((N_NODES, HIDDEN), jnp.float32),
        ],
        mesh=_sc_mesh(),
        compiler_params=_SC_PARAMS,
        scratch_types=[
            pltpu.VMEM_SHARED((N_NODES, HIDDEN), jnp.float32),
            pltpu.VMEM((2, K * ROW, HIDDEN), jnp.float32),
            pltpu.VMEM((2, K, ROW), jnp.int32),
            pltpu.SemaphoreType.DMA((2,)),
            pltpu.SemaphoreType.DMA((2,)),
        ],
    )
    return f(dA, dB, bA, bB, dst2, src2, zeros)


def _gather_half_body(h, td_hbm, tb_hbm, src2_hbm, dst2_hbm,
                      gd_hbm, gb_hbm, table, vbuf, ibuf, sem_i, sem_w):
    cid = lax.axis_index("c")
    sid = lax.axis_index("s")
    lo, hi = _wrange(sid, NS, HC)
    lo = lo + h * HC
    hi = hi + h * HC

    @pl.when(cid == 0)
    def _():
        _split_copy(td_hbm, table, sid, N_NODES)
        plsc.subcore_barrier()
        _gather_phase(table, src2_hbm, gd_hbm, vbuf, ibuf, sem_i, sem_w,
                      lo, hi, base=h * HC)

    @pl.when(cid == 1)
    def _():
        _split_copy(tb_hbm, table, sid, N_NODES)
        plsc.subcore_barrier()
        _gather_phase(table, dst2_hbm, gb_hbm, vbuf, ibuf, sem_i, sem_w,
                      lo, hi, base=h * HC)


@functools.partial(jax.jit, static_argnums=4)
def _sparse_gather_half(td, tb, src2, dst2, h):
    f = pl.kernel(
        functools.partial(_gather_half_body, h),
        out_type=[
            jax.ShapeDtypeStruct((N_EDGES // 2, HIDDEN), jnp.float32),
            jax.ShapeDtypeStruct((N_EDGES // 2, HIDDEN), jnp.float32),
        ],
        mesh=_sc_mesh(),
        compiler_params=_SC_PARAMS,
        scratch_types=[
            pltpu.VMEM_SHARED((N_NODES, HIDDEN), jnp.float32),
            pltpu.VMEM((2, K * ROW, HIDDEN), jnp.float32),
            pltpu.VMEM((2, K, ROW), jnp.int32),
            pltpu.SemaphoreType.DMA((2,)),
            pltpu.SemaphoreType.DMA((2,)),
        ],
    )
    return f(td, tb, src2, dst2)


def _incoming_body(dA_hbm, dB_hbm, dst2_hbm, zeros_hbm, t2_hbm,
                   table, vbuf, ibuf, sem_v, sem_i):
    cid = lax.axis_index("c")
    sid = lax.axis_index("s")
    _split_copy(zeros_hbm, table, sid, N_NODES)
    plsc.subcore_barrier()

    lo, hi = _wrange(sid, NS, HC)

    @pl.when(cid == 0)
    def _():
        _scatter_add_phase(dA_hbm, dst2_hbm, table, vbuf, ibuf, sem_v, sem_i,
                           lo, hi, base=0)

    @pl.when(cid == 1)
    def _():
        _scatter_add_phase(dB_hbm, dst2_hbm, table, vbuf, ibuf, sem_v, sem_i,
                           lo + HC, hi + HC, base=HC)
    plsc.subcore_barrier()

    _split_copy3(table, t2_hbm, cid, sid, N_NODES)


@jax.jit
def _sparse_incoming(dA, dB, dst2, zeros):
    f = pl.kernel(
        _incoming_body,
        out_type=jax.ShapeDtypeStruct((NC, N_NODES, HIDDEN), jnp.float32),
        mesh=_sc_mesh(),
        compiler_params=_SC_PARAMS,
        scratch_types=[
            pltpu.VMEM_SHARED((N_NODES, HIDDEN), jnp.float32),
            pltpu.VMEM((2, K * ROW, HIDDEN), jnp.float32),
            pltpu.VMEM((2, K, ROW), jnp.int32),
            pltpu.SemaphoreType.DMA((2,)),
            pltpu.SemaphoreType.DMA((2,)),
        ],
    )
    return f(dA, dB, dst2, zeros)


def _solv_body(sn_hbm, ssrc2_hbm, sdst2_hbm, zeros_hbm, t2_hbm,
               table, vbuf, ibuf, jbuf, sem_g, sem_i, sem_j):
    cid = lax.axis_index("c")
    sid = lax.axis_index("s")
    _split_copy(zeros_hbm, table, sid, N_SNODES)
    plsc.subcore_barrier()

    w = sid * NC + cid
    lo, hi = _wrange(w, NC * NS, N_SROWS)

    def fetch(c, slot):
        pltpu.async_copy(ssrc2_hbm.at[c], ibuf.at[slot], sem_i.at[slot])
        pltpu.async_copy(sdst2_hbm.at[c], jbuf.at[slot], sem_j.at[slot])

    def wait_fetch(slot):
        pltpu.make_async_copy(ssrc2_hbm.at[0], ibuf.at[slot],
                              sem_i.at[slot]).wait()
        pltpu.make_async_copy(sdst2_hbm.at[0], jbuf.at[slot],
                              sem_j.at[slot]).wait()

    def proc(slot):
        pltpu.sync_copy(sn_hbm.at[ibuf.at[slot, 0]], vbuf.at[slot])
        pltpu.sync_copy(vbuf.at[slot], table.at[jbuf.at[slot, 0]], add=True)

    @pl.when(lo < hi)
    def _():
        fetch(lo, 0)

    @pl.loop(lo, hi, step=2)
    def _(t):
        @pl.when(t + 1 < hi)
        def _():
            fetch(t + 1, 1)
        wait_fetch(0)
        proc(0)

        @pl.when(t + 2 < hi)
        def _():
            fetch(t + 2, 0)

        @pl.when(t + 1 < hi)
        def _():
            wait_fetch(1)
            proc(1)

    plsc.subcore_barrier()
    _split_copy3(table, t2_hbm, cid, sid, N_SNODES)


@jax.jit
def _sparse_solv(sn, ssrc2, sdst2, zeros):
    f = pl.kernel(
        _solv_body,
        out_type=jax.ShapeDtypeStruct((NC, N_SNODES, HIDDEN), jnp.float32),
        mesh=_sc_mesh(),
        compiler_params=_SC_PARAMS,
        scratch_types=[
            pltpu.VMEM_SHARED((N_SNODES, HIDDEN), jnp.float32),
            pltpu.VMEM((2, ROW, HIDDEN), jnp.float32),
            pltpu.VMEM((2, 1, ROW), jnp.int32),
            pltpu.VMEM((2, 1, ROW), jnp.int32),
            pltpu.SemaphoreType.DMA((2,)),
            pltpu.SemaphoreType.DMA((2,)),
            pltpu.SemaphoreType.DMA((2,)),
        ],
    )
    return f(sn, ssrc2, sdst2, zeros)


def _node_embed_body(nf_ref, mn_ref, mp_ref, cn_ref, cp_ref, nt_ref, p_ref):
    nf = nf_ref[...]
    nt_ref[...] = jnp.dot(nf, mn_ref[...], preferred_element_type=jnp.float32) + cn_ref[...]
    p_ref[...] = jnp.dot(nf, mp_ref[...], preferred_element_type=jnp.float32) + cp_ref[...]


def _node_embed(node_feats, m_node, m_p, c_node, c_p):
    n = node_feats.shape[0]
    return pl.pallas_call(
        _node_embed_body,
        grid=(n // NODE_BLK,),
        in_specs=[
            pl.BlockSpec((NODE_BLK, node_feats.shape[1]), lambda i: (i, 0)),
            pl.BlockSpec(m_node.shape, lambda i: (0, 0)),
            pl.BlockSpec(m_p.shape, lambda i: (0, 0)),
            pl.BlockSpec((1, HIDDEN), lambda i: (0, 0)),
            pl.BlockSpec((1, HIDDEN), lambda i: (0, 0)),
        ],
        out_specs=[
            pl.BlockSpec((NODE_BLK, HIDDEN), lambda i: (i, 0)),
            pl.BlockSpec((NODE_BLK, HIDDEN), lambda i: (i, 0)),
        ],
        out_shape=[
            jax.ShapeDtypeStruct((n, HIDDEN), jnp.float32),
            jax.ShapeDtypeStruct((n, HIDDEN), jnp.float32),
        ],
    )(node_feats, m_node, m_p, c_node, c_p)


def _init_body(ef_ref, gs_ref, gd_ref, me_ref, ce_ref, d_ref, b_ref):
    x = ef_ref[...]
    me = me_ref[...]
    ce = ce_ref[...]
    e = jnp.concatenate(
        [jnp.dot(x[:, :16], me, preferred_element_type=jnp.float32) + ce,
         jnp.dot(x[:, 16:], me, preferred_element_type=jnp.float32) + ce],
        axis=1)
    d_ref[...] = _leaky(gs_ref[...] + e)
    b_ref[...] = _leaky(gd_ref[...] + e)


def _init_edges(ef32, g_src, g_dst, m_edge, c_edge, h):
    n = g_src.shape[0]
    nb = n // PAIR_BLK
    return pl.pallas_call(
        _init_body,
        grid=(nb,),
        in_specs=[
            pl.BlockSpec((PAIR_BLK, 32), lambda i, h=h, nb=nb: (i + h * nb, 0)),
            pl.BlockSpec((PAIR_BLK, 128), lambda i: (i, 0)),
            pl.BlockSpec((PAIR_BLK, 128), lambda i: (i, 0)),
            pl.BlockSpec(m_edge.shape, lambda i: (0, 0)),
            pl.BlockSpec((1, HIDDEN), lambda i: (0, 0)),
        ],
        out_specs=[
            pl.BlockSpec((PAIR_BLK, 128), lambda i: (i, 0)),
            pl.BlockSpec((PAIR_BLK, 128), lambda i: (i, 0)),
        ],
        out_shape=[
            jax.ShapeDtypeStruct((n, 128), jnp.float32),
            jax.ShapeDtypeStruct((n, 128), jnp.float32),
        ],
    )(ef32, g_src, g_dst, m_edge, c_edge)


def _combine_body(gd_ref, gb_ref, d_ref, b_ref, w_ref, bias_ref, nd_ref, nb_ref):
    w = w_ref[...]
    d = d_ref[...]
    b = b_ref[...]
    bias = bias_ref[...]
    md = gd_ref[...] - b
    mb = gb_ref[...] - d
    nd_ref[...] = _leaky(jnp.dot(md, w, preferred_element_type=jnp.float32) + bias + d)
    nb_ref[...] = _leaky(jnp.dot(mb, w, preferred_element_type=jnp.float32) + bias + b)


def _combine(g_d, g_b, direct, backward, w2, bias2):
    n = direct.shape[0]
    return pl.pallas_call(
        _combine_body,
        grid=(n // PAIR_BLK,),
        in_specs=[
            pl.BlockSpec((PAIR_BLK, 128), lambda i: (i, 0)),
            pl.BlockSpec((PAIR_BLK, 128), lambda i: (i, 0)),
            pl.BlockSpec((PAIR_BLK, 128), lambda i: (i, 0)),
            pl.BlockSpec((PAIR_BLK, 128), lambda i: (i, 0)),
            pl.BlockSpec((128, 128), lambda i: (0, 0)),
            pl.BlockSpec((1, 128), lambda i: (0, 0)),
        ],
        out_specs=[
            pl.BlockSpec((PAIR_BLK, 128), lambda i: (i, 0)),
            pl.BlockSpec((PAIR_BLK, 128), lambda i: (i, 0)),
        ],
        out_shape=[
            jax.ShapeDtypeStruct((n, 128), jnp.float32),
            jax.ShapeDtypeStruct((n, 128), jnp.float32),
        ],
    )(g_d, g_b, direct, backward, w2, bias2)


def _readout_body(p_ref, inc_ref, wa_ref, sum_ref):
    i = pl.program_id(0)
    inc = inc_ref[0] + inc_ref[1]
    h = _leaky(p_ref[...] + jnp.dot(inc, wa_ref[...],
                                    preferred_element_type=jnp.float32))
    s = jnp.sum(h, axis=0, keepdims=True)

    @pl.when(i == 0)
    def _():
        sum_ref[...] = jnp.zeros_like(sum_ref)

    sum_ref[...] += s


def _readout(p, inc2, wa_i):
    n = p.shape[0]
    return pl.pallas_call(
        _readout_body,
        grid=(n // NODE_BLK,),
        in_specs=[
            pl.BlockSpec((NODE_BLK, HIDDEN), lambda i: (i, 0)),
            pl.BlockSpec((NC, NODE_BLK, HIDDEN), lambda i: (0, i, 0)),
            pl.BlockSpec((HIDDEN, HIDDEN), lambda i: (0, 0)),
        ],
        out_specs=pl.BlockSpec((1, HIDDEN), lambda i: (0, 0)),
        out_shape=jax.ShapeDtypeStruct((1, HIDDEN), jnp.float32),
    )(p, inc2, wa_i)


def _solv_embed_body(sf_ref, w_ref, b_ref, sn_ref):
    sn_ref[...] = _leaky(
        jnp.dot(sf_ref[...], w_ref[...], preferred_element_type=jnp.float32)
        + b_ref[...])


def _solv_embed(solv_node_feats, w_solv, b_solv):
    n = solv_node_feats.shape[0]
    blk = 1000
    return pl.pallas_call(
        _solv_embed_body,
        grid=(n // blk,),
        in_specs=[
            pl.BlockSpec((blk, solv_node_feats.shape[1]), lambda i: (i, 0)),
            pl.BlockSpec(w_solv.shape, lambda i: (0, 0)),
            pl.BlockSpec((1, HIDDEN), lambda i: (0, 0)),
        ],
        out_specs=pl.BlockSpec((blk, HIDDEN), lambda i: (i, 0)),
        out_shape=jax.ShapeDtypeStruct((n, HIDDEN), jnp.float32),
    )(solv_node_feats, w_solv, b_solv.reshape(1, HIDDEN))


def _solv_final_body(sn_ref, agg_ref, sum_ref):
    i = pl.program_id(0)
    h = _leaky(sn_ref[...] + agg_ref[0] + agg_ref[1])
    s = jnp.sum(h, axis=0, keepdims=True)

    @pl.when(i == 0)
    def _():
        sum_ref[...] = jnp.zeros_like(sum_ref)

    sum_ref[...] += s


def _solv_final(sn, agg2):
    n = sn.shape[0]
    blk = 1000
    return pl.pallas_call(
        _solv_final_body,
        grid=(n // blk,),
        in_specs=[
            pl.BlockSpec((blk, HIDDEN), lambda i: (i, 0)),
            pl.BlockSpec((NC, blk, HIDDEN), lambda i: (0, i, 0)),
        ],
        out_specs=pl.BlockSpec((1, HIDDEN), lambda i: (0, 0)),
        out_shape=jax.ShapeDtypeStruct((1, HIDDEN), jnp.float32),
    )(sn, agg2)


def kernel(graph, node_feats, edge_feats, solv_graph, solv_node_feats,
           W_node, b_node, W_edge, b_edge, W_init, b_init, W_a, b_a,
           W_layers, b_layers, W_solv, b_solv, W_out, b_out):
    src2 = graph[0].reshape(N_ROWS // K, K, ROW)
    dst2 = graph[1].reshape(N_ROWS // K, K, ROW)
    ssrc2 = solv_graph[0].reshape(N_SROWS, 1, ROW)
    sdst2 = solv_graph[1].reshape(N_SROWS, 1, ROW)
    zeros = jnp.zeros((N_NODES, HIDDEN), jnp.float32)

    wi_h = W_init[:HIDDEN]
    wi_e = W_init[HIDDEN:]
    wa_n = W_a[:HIDDEN]
    wa_i = W_a[HIDDEN:]
    m_node = W_node @ wi_h
    m_p = W_node @ wa_n
    c_node = (b_node @ wi_h).reshape(1, HIDDEN)
    c_p = (b_node @ wa_n + b_a).reshape(1, HIDDEN)
    m_edge = W_edge @ wi_e
    c_edge = (b_edge @ wi_e + b_init).reshape(1, HIDDEN)

    nt, p = _node_embed(node_feats, m_node, m_p, c_node, c_p)

    HPAIR = N_PAIRS // 2
    HE = N_EDGES // 2
    gsA, gdA, gsB, gdB = _sparse_init_gather(nt, src2, dst2)
    ef32 = edge_feats.reshape(N_PAIRS, 32)
    dA, bA = _init_edges(ef32, gsA.reshape(HPAIR, 128),
                         gdA.reshape(HPAIR, 128), m_edge, c_edge, 0)
    dB, bB = _init_edges(ef32, gsB.reshape(HPAIR, 128),
                         gdB.reshape(HPAIR, 128), m_edge, c_edge, 1)

    zeros128 = jnp.zeros((128, 128), jnp.float32)
    for l in range(LAYERS):
        w = W_layers[l]
        w2 = zeros128.at[:HIDDEN, :HIDDEN].set(w).at[HIDDEN:, HIDDEN:].set(w)
        bias2 = jnp.tile(b_layers[l], 2).reshape(1, 128)
        td, tb = _sparse_seg(dA.reshape(HE, HIDDEN), dB.reshape(HE, HIDDEN),
                             bA.reshape(HE, HIDDEN), bB.reshape(HE, HIDDEN),
                             dst2, src2, zeros)
        g_dA, g_bA = _sparse_gather_half(td, tb, src2, dst2, 0)
        g_dB, g_bB = _sparse_gather_half(td, tb, src2, dst2, 1)
        dA, bA = _combine(g_dA.reshape(HPAIR, 128), g_bA.reshape(HPAIR, 128),
                          dA, bA, w2, bias2)
        dB, bB = _combine(g_dB.reshape(HPAIR, 128), g_bB.reshape(HPAIR, 128),
                          dB, bB, w2, bias2)

    inc2 = _sparse_incoming(dA.reshape(HE, HIDDEN), dB.reshape(HE, HIDDEN),
                            dst2, zeros)
    solute_sum = _readout(p, inc2, wa_i)

    sn = _solv_embed(solv_node_feats, W_solv, b_solv)
    agg2 = _sparse_solv(sn, ssrc2, sdst2, zeros)
    solv_sum = _solv_final(sn, agg2)

    solute_pool = solute_sum[0] / node_feats.shape[0]
    solv_pool = solv_sum[0] / solv_node_feats.shape[0]
    out = jnp.concatenate([solute_pool, solv_pool], axis=-1) @ W_out + b_out
    return out


def _init_gather_body(nt_hbm, src2_hbm, dst2_hbm,
                      gsA_hbm, gdA_hbm, gsB_hbm, gdB_hbm,
                      table, vbuf, ibuf, sem_i, sem_w):
    cid = lax.axis_index("c")
    sid = lax.axis_index("s")
    _split_copy(nt_hbm, table, sid, N_NODES)
    plsc.subcore_barrier()

    lo, hi = _wrange(sid, NS, HC)

    @pl.when(cid == 0)
    def _():
        _gather_phase(table, src2_hbm, gsA_hbm, vbuf, ibuf, sem_i, sem_w,
                      lo, hi, base=0)
        _gather_phase(table, src2_hbm, gsB_hbm, vbuf, ibuf, sem_i, sem_w,
                      lo + HC, hi + HC, base=HC)

    @pl.when(cid == 1)
    def _():
        _gather_phase(table, dst2_hbm, gdA_hbm, vbuf, ibuf, sem_i, sem_w,
                      lo, hi, base=0)
        _gather_phase(table, dst2_hbm, gdB_hbm, vbuf, ibuf, sem_i, sem_w,
                      lo + HC, hi + HC, base=HC)


@jax.jit
def _sparse_init_gather(nt, src2, dst2):
    f = pl.kernel(
        _init_gather_body,
        out_type=[
            jax.ShapeDtypeStruct((N_EDGES // 2, HIDDEN), jnp.float32),
            jax.ShapeDtypeStruct((N_EDGES // 2, HIDDEN), jnp.float32),
            jax.ShapeDtypeStruct((N_EDGES // 2, HIDDEN), jnp.float32),
            jax.ShapeDtypeStruct((N_EDGES // 2, HIDDEN), jnp.float32),
        ],
        mesh=_sc_mesh(),
        compiler_params=_SC_PARAMS,
        scratch_types=[
            pltpu.VMEM_SHARED((N_NODES, HIDDEN), jnp.float32),
            pltpu.VMEM((2, K * ROW, HIDDEN), jnp.float32),
            pltpu.VMEM((2, K, ROW), jnp.int32),
            pltpu.SemaphoreType.DMA((2,)),
            pltpu.SemaphoreType.DMA((2,)),
        ],
    )
    return f(nt, src2, dst2)


# speedup vs baseline: 5.7947x; 1.0116x over previous
import functools

import jax
import jax.numpy as jnp
from jax import lax
from jax.experimental import pallas as pl
from jax.experimental.pallas import tpu as pltpu
from jax.experimental.pallas import tpu_sc as plsc

HIDDEN = 64
LAYERS = 4
N_NODES = 10000
N_EDGES = 320000
N_SNODES = 5000
N_SEDGES = 80000

PAIR_BLK = 4000
NODE_BLK = 2000
N_PAIRS = 160000

ROW = 128
K = 2
NC = 2
NS = 16
N_ROWS = N_EDGES // ROW
N_SROWS = N_SEDGES // ROW

_SC_PARAMS = pltpu.CompilerParams(use_tc_tiling_on_sc=False)


@functools.lru_cache(maxsize=None)
def _sc_mesh():
    return plsc.VectorSubcoreMesh(core_axis_name="c", subcore_axis_name="s")


def _leaky(x):
    return jnp.where(x > 0, x, 0.01 * x)


def _wrange(w, nw, count):
    per = count // nw
    rem = count % nw
    lo = w * per + jnp.minimum(w, rem)
    hi = lo + per + jnp.where(w < rem, 1, 0)
    return lo, hi


def _split_copy(src, dst, sid, rows):
    per = (rows // NS) // 8 * 8
    tail = rows - per * NS
    pltpu.sync_copy(src.at[pl.ds(sid * per, per)],
                    dst.at[pl.ds(sid * per, per)])
    if tail:
        @pl.when(sid == NS - 1)
        def _():
            pltpu.sync_copy(src.at[pl.ds(per * NS, tail)],
                            dst.at[pl.ds(per * NS, tail)])


def _split_copy3(src, dst3, cid, sid, rows):
    per = (rows // NS) // 8 * 8
    tail = rows - per * NS
    pltpu.sync_copy(src.at[pl.ds(sid * per, per)],
                    dst3.at[cid, pl.ds(sid * per, per)])
    if tail:
        @pl.when(sid == NS - 1)
        def _():
            pltpu.sync_copy(src.at[pl.ds(per * NS, tail)],
                            dst3.at[cid, pl.ds(per * NS, tail)])


def _scatter_add_phase(vals_hbm, idx_hbm, table, vbuf, ibuf, sem_v, sem_i,
                       lo, hi, base=0):
    def fetch(c, slot):
        pltpu.async_copy(vals_hbm.at[pl.ds((c - base) * (K * ROW), K * ROW)],
                         vbuf.at[slot], sem_v.at[slot])
        pltpu.async_copy(idx_hbm.at[c], ibuf.at[slot], sem_i.at[slot])

    def wait_fetch(slot):
        pltpu.make_async_copy(vals_hbm.at[pl.ds(0, K * ROW)],
                              vbuf.at[slot], sem_v.at[slot]).wait()
        pltpu.make_async_copy(idx_hbm.at[0], ibuf.at[slot],
                              sem_i.at[slot]).wait()

    def scat(slot):
        for j in range(K):
            pltpu.sync_copy(vbuf.at[slot, pl.ds(j * ROW, ROW)],
                            table.at[ibuf.at[slot, j]], add=True)

    @pl.when(lo < hi)
    def _():
        fetch(lo, 0)

    @pl.loop(lo, hi, step=2)
    def _(t):
        @pl.when(t + 1 < hi)
        def _():
            fetch(t + 1, 1)
        wait_fetch(0)
        scat(0)

        @pl.when(t + 2 < hi)
        def _():
            fetch(t + 2, 0)

        @pl.when(t + 1 < hi)
        def _():
            wait_fetch(1)
            scat(1)


def _gather_phase(table, gidx_hbm, gout_hbm, vbuf, ibuf, sem_i, sem_w,
                  lo, hi, base=0):
    def fetch(c, slot):
        pltpu.async_copy(gidx_hbm.at[c], ibuf.at[slot], sem_i.at[slot])

    def wait_fetch(slot):
        pltpu.make_async_copy(gidx_hbm.at[0], ibuf.at[slot],
                              sem_i.at[slot]).wait()

    def wait_write(slot):
        pltpu.make_async_copy(vbuf.at[slot],
                              gout_hbm.at[pl.ds(0, K * ROW)],
                              sem_w.at[slot]).wait()

    def proc(c, t, slot, first_use_cond):
        @pl.when(first_use_cond)
        def _():
            wait_write(slot)
        for j in range(K):
            pltpu.sync_copy(table.at[ibuf.at[slot, j]],
                            vbuf.at[slot, pl.ds(j * ROW, ROW)])
        pltpu.async_copy(vbuf.at[slot],
                         gout_hbm.at[pl.ds((c - base) * (K * ROW), K * ROW)],
                         sem_w.at[slot])

    @pl.when(lo < hi)
    def _():
        fetch(lo, 0)

    @pl.loop(lo, hi, step=2)
    def _(t):
        @pl.when(t + 1 < hi)
        def _():
            fetch(t + 1, 1)
        wait_fetch(0)
        proc(t, t, 0, t >= lo + 2)

        @pl.when(t + 2 < hi)
        def _():
            fetch(t + 2, 0)

        @pl.when(t + 1 < hi)
        def _():
            wait_fetch(1)
            proc(t + 1, t, 1, t >= lo + 1)

    @pl.when(lo < hi)
    def _():
        wait_write(0)

    @pl.when(lo + 1 < hi)
    def _():
        wait_write(1)


HC = (N_EDGES // ROW) // 2 // K


def _seg_chain(valsA, valsB, idx_hbm, tout_hbm, zeros_hbm,
               table, vbuf, ibuf, sem_v, sem_i, sid):
    _split_copy(zeros_hbm, table, sid, N_NODES)
    plsc.subcore_barrier()

    lo, hi = _wrange(sid, NS, 2 * HC)
    _scatter_add_phase(valsA, idx_hbm, table, vbuf, ibuf, sem_v, sem_i,
                       lo, jnp.minimum(hi, HC), base=0)
    _scatter_add_phase(valsB, idx_hbm, table, vbuf, ibuf, sem_v, sem_i,
                       jnp.maximum(lo, HC), hi, base=HC)
    plsc.subcore_barrier()

    _split_copy(table, tout_hbm, sid, N_NODES)


def _seg_body(dA_hbm, dB_hbm, bA_hbm, bB_hbm, dst2_hbm, src2_hbm, zeros_hbm,
              td_hbm, tb_hbm, table, vbuf, ibuf, sem_v, sem_i):
    cid = lax.axis_index("c")
    sid = lax.axis_index("s")

    @pl.when(cid == 0)
    def _():
        _seg_chain(dA_hbm, dB_hbm, dst2_hbm, td_hbm, zeros_hbm,
                   table, vbuf, ibuf, sem_v, sem_i, sid)

    @pl.when(cid == 1)
    def _():
        _seg_chain(bA_hbm, bB_hbm, src2_hbm, tb_hbm, zeros_hbm,
                   table, vbuf, ibuf, sem_v, sem_i, sid)


@jax.jit
def _sparse_seg(dA, dB, bA, bB, dst2, src2, zeros):
    f = pl.kernel(
        _seg_body,
        out_type=[
            jax.ShapeDtypeStruct((N_NODES, HIDDEN), jnp.float32),
            jax.ShapeDtypeStruct((N_NODES, HIDDEN), jnp.float32),
        ],
        mesh=_sc_mesh(),
        compiler_params=_SC_PARAMS,
        scratch_types=[
            pltpu.VMEM_SHARED((N_NODES, HIDDEN), jnp.float32),
            pltpu.VMEM((2, K * ROW, HIDDEN), jnp.float32),
            pltpu.VMEM((2, K, ROW), jnp.int32),
            pltpu.SemaphoreType.DMA((2,)),
            pltpu.SemaphoreType.DMA((2,)),
        ],
    )
    return f(dA, dB, bA, bB, dst2, src2, zeros)


def _gather_half_body(h, td_hbm, tb_hbm, src2_hbm, dst2_hbm,
                      gd_hbm, gb_hbm, table, vbuf, ibuf, sem_i, sem_w):
    cid = lax.axis_index("c")
    sid = lax.axis_index("s")
    lo, hi = _wrange(sid, NS, HC)
    lo = lo + h * HC
    hi = hi + h * HC

    @pl.when(cid == 0)
    def _():
        _split_copy(td_hbm, table, sid, N_NODES)
        plsc.subcore_barrier()
        _gather_phase(table, src2_hbm, gd_hbm, vbuf, ibuf, sem_i, sem_w,
                      lo, hi, base=h * HC)

    @pl.when(cid == 1)
    def _():
        _split_copy(tb_hbm, table, sid, N_NODES)
        plsc.subcore_barrier()
        _gather_phase(table, dst2_hbm, gb_hbm, vbuf, ibuf, sem_i, sem_w,
                      lo, hi, base=h * HC)


@functools.partial(jax.jit, static_argnums=4)
def _sparse_gather_half(td, tb, src2, dst2, h):
    f = pl.kernel(
        functools.partial(_gather_half_body, h),
        out_type=[
            jax.ShapeDtypeStruct((N_EDGES // 2, HIDDEN), jnp.float32),
            jax.ShapeDtypeStruct((N_EDGES // 2, HIDDEN), jnp.float32),
        ],
        mesh=_sc_mesh(),
        compiler_params=_SC_PARAMS,
        scratch_types=[
            pltpu.VMEM_SHARED((N_NODES, HIDDEN), jnp.float32),
            pltpu.VMEM((2, K * ROW, HIDDEN), jnp.float32),
            pltpu.VMEM((2, K, ROW), jnp.int32),
            pltpu.SemaphoreType.DMA((2,)),
            pltpu.SemaphoreType.DMA((2,)),
        ],
    )
    return f(td, tb, src2, dst2)


def _incoming_body(dA_hbm, dB_hbm, dst2_hbm, zeros_hbm, t2_hbm,
                   table, vbuf, ibuf, sem_v, sem_i):
    cid = lax.axis_index("c")
    sid = lax.axis_index("s")
    _split_copy(zeros_hbm, table, sid, N_NODES)
    plsc.subcore_barrier()

    lo, hi = _wrange(sid, NS, HC)

    @pl.when(cid == 0)
    def _():
        _scatter_add_phase(dA_hbm, dst2_hbm, table, vbuf, ibuf, sem_v, sem_i,
                           lo, hi, base=0)

    @pl.when(cid == 1)
    def _():
        _scatter_add_phase(dB_hbm, dst2_hbm, table, vbuf, ibuf, sem_v, sem_i,
                           lo + HC, hi + HC, base=HC)
    plsc.subcore_barrier()

    _split_copy3(table, t2_hbm, cid, sid, N_NODES)


@jax.jit
def _sparse_incoming(dA, dB, dst2, zeros):
    f = pl.kernel(
        _incoming_body,
        out_type=jax.ShapeDtypeStruct((NC, N_NODES, HIDDEN), jnp.float32),
        mesh=_sc_mesh(),
        compiler_params=_SC_PARAMS,
        scratch_types=[
            pltpu.VMEM_SHARED((N_NODES, HIDDEN), jnp.float32),
            pltpu.VMEM((2, K * ROW, HIDDEN), jnp.float32),
            pltpu.VMEM((2, K, ROW), jnp.int32),
            pltpu.SemaphoreType.DMA((2,)),
            pltpu.SemaphoreType.DMA((2,)),
        ],
    )
    return f(dA, dB, dst2, zeros)


def _solv_body(sn_hbm, ssrc2_hbm, sdst2_hbm, zeros_hbm, t2_hbm,
               table, vbuf, ibuf, jbuf, sem_g, sem_i, sem_j):
    cid = lax.axis_index("c")
    sid = lax.axis_index("s")
    _split_copy(zeros_hbm, table, sid, N_SNODES)
    plsc.subcore_barrier()

    w = sid * NC + cid
    lo, hi = _wrange(w, NC * NS, N_SROWS)

    def fetch(c, slot):
        pltpu.async_copy(ssrc2_hbm.at[c], ibuf.at[slot], sem_i.at[slot])
        pltpu.async_copy(sdst2_hbm.at[c], jbuf.at[slot], sem_j.at[slot])

    def wait_fetch(slot):
        pltpu.make_async_copy(ssrc2_hbm.at[0], ibuf.at[slot],
                              sem_i.at[slot]).wait()
        pltpu.make_async_copy(sdst2_hbm.at[0], jbuf.at[slot],
                              sem_j.at[slot]).wait()

    def proc(slot):
        pltpu.sync_copy(sn_hbm.at[ibuf.at[slot, 0]], vbuf.at[slot])
        pltpu.sync_copy(vbuf.at[slot], table.at[jbuf.at[slot, 0]], add=True)

    @pl.when(lo < hi)
    def _():
        fetch(lo, 0)

    @pl.loop(lo, hi, step=2)
    def _(t):
        @pl.when(t + 1 < hi)
        def _():
            fetch(t + 1, 1)
        wait_fetch(0)
        proc(0)

        @pl.when(t + 2 < hi)
        def _():
            fetch(t + 2, 0)

        @pl.when(t + 1 < hi)
        def _():
            wait_fetch(1)
            proc(1)

    plsc.subcore_barrier()
    _split_copy3(table, t2_hbm, cid, sid, N_SNODES)


@jax.jit
def _sparse_solv(sn, ssrc2, sdst2, zeros):
    f = pl.kernel(
        _solv_body,
        out_type=jax.ShapeDtypeStruct((NC, N_SNODES, HIDDEN), jnp.float32),
        mesh=_sc_mesh(),
        compiler_params=_SC_PARAMS,
        scratch_types=[
            pltpu.VMEM_SHARED((N_SNODES, HIDDEN), jnp.float32),
            pltpu.VMEM((2, ROW, HIDDEN), jnp.float32),
            pltpu.VMEM((2, 1, ROW), jnp.int32),
            pltpu.VMEM((2, 1, ROW), jnp.int32),
            pltpu.SemaphoreType.DMA((2,)),
            pltpu.SemaphoreType.DMA((2,)),
            pltpu.SemaphoreType.DMA((2,)),
        ],
    )
    return f(sn, ssrc2, sdst2, zeros)


def _node_embed_body(nf_ref, mn_ref, mp_ref, cn_ref, cp_ref, nt_ref, p_ref):
    nf = nf_ref[...]
    nt_ref[...] = jnp.dot(nf, mn_ref[...], preferred_element_type=jnp.float32) + cn_ref[...]
    p_ref[...] = jnp.dot(nf, mp_ref[...], preferred_element_type=jnp.float32) + cp_ref[...]


def _node_embed(node_feats, m_node, m_p, c_node, c_p):
    n = node_feats.shape[0]
    return pl.pallas_call(
        _node_embed_body,
        grid=(n // NODE_BLK,),
        in_specs=[
            pl.BlockSpec((NODE_BLK, node_feats.shape[1]), lambda i: (i, 0)),
            pl.BlockSpec(m_node.shape, lambda i: (0, 0)),
            pl.BlockSpec(m_p.shape, lambda i: (0, 0)),
            pl.BlockSpec((1, HIDDEN), lambda i: (0, 0)),
            pl.BlockSpec((1, HIDDEN), lambda i: (0, 0)),
        ],
        out_specs=[
            pl.BlockSpec((NODE_BLK, HIDDEN), lambda i: (i, 0)),
            pl.BlockSpec((NODE_BLK, HIDDEN), lambda i: (i, 0)),
        ],
        out_shape=[
            jax.ShapeDtypeStruct((n, HIDDEN), jnp.float32),
            jax.ShapeDtypeStruct((n, HIDDEN), jnp.float32),
        ],
    )(node_feats, m_node, m_p, c_node, c_p)


def _init_body(ef_ref, gs_ref, gd_ref, me_ref, ce_ref, d_ref, b_ref):
    x = ef_ref[...]
    me = me_ref[...]
    ce = ce_ref[...]
    e = jnp.concatenate(
        [jnp.dot(x[:, :16], me, preferred_element_type=jnp.float32) + ce,
         jnp.dot(x[:, 16:], me, preferred_element_type=jnp.float32) + ce],
        axis=1)
    d_ref[...] = _leaky(gs_ref[...] + e)
    b_ref[...] = _leaky(gd_ref[...] + e)


def _init_edges(ef32, g_src, g_dst, m_edge, c_edge, h):
    n = g_src.shape[0]
    nb = n // PAIR_BLK
    return pl.pallas_call(
        _init_body,
        grid=(nb,),
        in_specs=[
            pl.BlockSpec((PAIR_BLK, 32), lambda i, h=h, nb=nb: (i + h * nb, 0)),
            pl.BlockSpec((PAIR_BLK, 128), lambda i: (i, 0)),
            pl.BlockSpec((PAIR_BLK, 128), lambda i: (i, 0)),
            pl.BlockSpec(m_edge.shape, lambda i: (0, 0)),
            pl.BlockSpec((1, HIDDEN), lambda i: (0, 0)),
        ],
        out_specs=[
            pl.BlockSpec((PAIR_BLK, 128), lambda i: (i, 0)),
            pl.BlockSpec((PAIR_BLK, 128), lambda i: (i, 0)),
        ],
        out_shape=[
            jax.ShapeDtypeStruct((n, 128), jnp.float32),
            jax.ShapeDtypeStruct((n, 128), jnp.float32),
        ],
    )(ef32, g_src, g_dst, m_edge, c_edge)


def _combine_body(gd_ref, gb_ref, d_ref, b_ref, w_ref, bias_ref, nd_ref, nb_ref):
    w = w_ref[...]
    d = d_ref[...]
    b = b_ref[...]
    bias = bias_ref[...]
    md = gd_ref[...] - b
    mb = gb_ref[...] - d
    nd_ref[...] = _leaky(jnp.dot(md, w, preferred_element_type=jnp.float32) + bias + d)
    nb_ref[...] = _leaky(jnp.dot(mb, w, preferred_element_type=jnp.float32) + bias + b)


def _combine(g_d, g_b, direct, backward, w2, bias2):
    n = direct.shape[0]
    return pl.pallas_call(
        _combine_body,
        grid=(n // PAIR_BLK,),
        in_specs=[
            pl.BlockSpec((PAIR_BLK, 128), lambda i: (i, 0)),
            pl.BlockSpec((PAIR_BLK, 128), lambda i: (i, 0)),
            pl.BlockSpec((PAIR_BLK, 128), lambda i: (i, 0)),
            pl.BlockSpec((PAIR_BLK, 128), lambda i: (i, 0)),
            pl.BlockSpec((128, 128), lambda i: (0, 0)),
            pl.BlockSpec((1, 128), lambda i: (0, 0)),
        ],
        out_specs=[
            pl.BlockSpec((PAIR_BLK, 128), lambda i: (i, 0)),
            pl.BlockSpec((PAIR_BLK, 128), lambda i: (i, 0)),
        ],
        out_shape=[
            jax.ShapeDtypeStruct((n, 128), jnp.float32),
            jax.ShapeDtypeStruct((n, 128), jnp.float32),
        ],
    )(g_d, g_b, direct, backward, w2, bias2)


def _readout_body(p_ref, inc_ref, wa_ref, sum_ref):
    i = pl.program_id(0)
    inc = inc_ref[0] + inc_ref[1]
    h = _leaky(p_ref[...] + jnp.dot(inc, wa_ref[...],
                                    preferred_element_type=jnp.float32))
    s = jnp.sum(h, axis=0, keepdims=True)

    @pl.when(i == 0)
    def _():
        sum_ref[...] = jnp.zeros_like(sum_ref)

    sum_ref[...] += s


def _readout(p, inc2, wa_i):
    n = p.shape[0]
    return pl.pallas_call(
        _readout_body,
        grid=(n // NODE_BLK,),
        in_specs=[
            pl.BlockSpec((NODE_BLK, HIDDEN), lambda i: (i, 0)),
            pl.BlockSpec((NC, NODE_BLK, HIDDEN), lambda i: (0, i, 0)),
            pl.BlockSpec((HIDDEN, HIDDEN), lambda i: (0, 0)),
        ],
        out_specs=pl.BlockSpec((1, HIDDEN), lambda i: (0, 0)),
        out_shape=jax.ShapeDtypeStruct((1, HIDDEN), jnp.float32),
    )(p, inc2, wa_i)


def _solv_embed_body(sf_ref, w_ref, b_ref, sn_ref):
    sn_ref[...] = _leaky(
        jnp.dot(sf_ref[...], w_ref[...], preferred_element_type=jnp.float32)
        + b_ref[...])


def _solv_embed(solv_node_feats, w_solv, b_solv):
    n = solv_node_feats.shape[0]
    blk = 1000
    return pl.pallas_call(
        _solv_embed_body,
        grid=(n // blk,),
        in_specs=[
            pl.BlockSpec((blk, solv_node_feats.shape[1]), lambda i: (i, 0)),
            pl.BlockSpec(w_solv.shape, lambda i: (0, 0)),
            pl.BlockSpec((1, HIDDEN), lambda i: (0, 0)),
        ],
        out_specs=pl.BlockSpec((blk, HIDDEN), lambda i: (i, 0)),
        out_shape=jax.ShapeDtypeStruct((n, HIDDEN), jnp.float32),
    )(solv_node_feats, w_solv, b_solv.reshape(1, HIDDEN))


def _solv_final_body(sn_ref, agg_ref, sum_ref):
    i = pl.program_id(0)
    h = _leaky(sn_ref[...] + agg_ref[0] + agg_ref[1])
    s = jnp.sum(h, axis=0, keepdims=True)

    @pl.when(i == 0)
    def _():
        sum_ref[...] = jnp.zeros_like(sum_ref)

    sum_ref[...] += s


def _solv_final(sn, agg2):
    n = sn.shape[0]
    blk = 1000
    return pl.pallas_call(
        _solv_final_body,
        grid=(n // blk,),
        in_specs=[
            pl.BlockSpec((blk, HIDDEN), lambda i: (i, 0)),
            pl.BlockSpec((NC, blk, HIDDEN), lambda i: (0, i, 0)),
        ],
        out_specs=pl.BlockSpec((1, HIDDEN), lambda i: (0, 0)),
        out_shape=jax.ShapeDtypeStruct((1, HIDDEN), jnp.float32),
    )(sn, agg2)


def kernel(graph, node_feats, edge_feats, solv_graph, solv_node_feats,
           W_node, b_node, W_edge, b_edge, W_init, b_init, W_a, b_a,
           W_layers, b_layers, W_solv, b_solv, W_out, b_out):
    src2 = graph[0].reshape(N_ROWS // K, K, ROW)
    dst2 = graph[1].reshape(N_ROWS // K, K, ROW)
    ssrc2 = solv_graph[0].reshape(N_SROWS, 1, ROW)
    sdst2 = solv_graph[1].reshape(N_SROWS, 1, ROW)
    zeros = jnp.zeros((N_NODES, HIDDEN), jnp.float32)

    wi_h = W_init[:HIDDEN]
    wi_e = W_init[HIDDEN:]
    wa_n = W_a[:HIDDEN]
    wa_i = W_a[HIDDEN:]
    m_node = W_node @ wi_h
    m_p = W_node @ wa_n
    c_node = (b_node @ wi_h).reshape(1, HIDDEN)
    c_p = (b_node @ wa_n + b_a).reshape(1, HIDDEN)
    m_edge = W_edge @ wi_e
    c_edge = (b_edge @ wi_e + b_init).reshape(1, HIDDEN)

    nt, p = _node_embed(node_feats, m_node, m_p, c_node, c_p)

    HPAIR = N_PAIRS // 2
    HE = N_EDGES // 2
    gsA, gdA, gsB, gdB = _sparse_init_gather(nt, src2, dst2)
    ef32 = edge_feats.reshape(N_PAIRS, 32)
    dA, bA = _init_edges(ef32, gsA.reshape(HPAIR, 128),
                         gdA.reshape(HPAIR, 128), m_edge, c_edge, 0)
    dB, bB = _init_edges(ef32, gsB.reshape(HPAIR, 128),
                         gdB.reshape(HPAIR, 128), m_edge, c_edge, 1)

    zeros128 = jnp.zeros((128, 128), jnp.float32)
    for l in range(LAYERS):
        w = W_layers[l]
        w2 = zeros128.at[:HIDDEN, :HIDDEN].set(w).at[HIDDEN:, HIDDEN:].set(w)
        bias2 = jnp.tile(b_layers[l], 2).reshape(1, 128)
        td, tb = _sparse_seg(dA.reshape(HE, HIDDEN), dB.reshape(HE, HIDDEN),
                             bA.reshape(HE, HIDDEN), bB.reshape(HE, HIDDEN),
                             dst2, src2, zeros)
        g_dA, g_bA = _sparse_gather_half(td, tb, src2, dst2, 0)
        g_dB, g_bB = _sparse_gather_half(td, tb, src2, dst2, 1)
        dA, bA = _combine(g_dA.reshape(HPAIR, 128), g_bA.reshape(HPAIR, 128),
                          dA, bA, w2, bias2)
        dB, bB = _combine(g_dB.reshape(HPAIR, 128), g_bB.reshape(HPAIR, 128),
                          dB, bB, w2, bias2)

    inc2 = _sparse_incoming(dA.reshape(HE, HIDDEN), dB.reshape(HE, HIDDEN),
                            dst2, zeros)
    solute_sum = _readout(p, inc2, wa_i)

    sn = _solv_embed(solv_node_feats, W_solv, b_solv)
    agg2 = _sparse_solv(sn, ssrc2, sdst2, zeros)
    solv_sum = _solv_final(sn, agg2)

    solute_pool = solute_sum[0] / node_feats.shape[0]
    solv_pool = solv_sum[0] / solv_node_feats.shape[0]
    out = jnp.concatenate([solute_pool, solv_pool], axis=-1) @ W_out + b_out
    return out


def _init_gather_body(nt_hbm, src2_hbm, dst2_hbm,
                      gsA_hbm, gdA_hbm, gsB_hbm, gdB_hbm,
                      table, vbuf, ibuf, sem_i, sem_w):
    cid = lax.axis_index("c")
    sid = lax.axis_index("s")
    _split_copy(nt_hbm, table, sid, N_NODES)
    plsc.subcore_barrier()

    lo, hi = _wrange(sid, NS, HC)

    @pl.when(cid == 0)
    def _():
        _gather_phase(table, src2_hbm, gsA_hbm, vbuf, ibuf, sem_i, sem_w,
                      lo, hi, base=0)
        _gather_phase(table, src2_hbm, gsB_hbm, vbuf, ibuf, sem_i, sem_w,
                      lo + HC, hi + HC, base=HC)

    @pl.when(cid == 1)
    def _():
        _gather_phase(table, dst2_hbm, gdA_hbm, vbuf, ibuf, sem_i, sem_w,
                      lo, hi, base=0)
        _gather_phase(table, dst2_hbm, gdB_hbm, vbuf, ibuf, sem_i, sem_w,
                      lo + HC, hi + HC, base=HC)


@jax.jit
def _sparse_init_gather(nt, src2, dst2):
    f = pl.kernel(
        _init_gather_body,
        out_type=[
            jax.ShapeDtypeStruct((N_EDGES // 2, HIDDEN), jnp.float32),
            jax.ShapeDtypeStruct((N_EDGES // 2, HIDDEN), jnp.float32),
            jax.ShapeDtypeStruct((N_EDGES // 2, HIDDEN), jnp.float32),
            jax.ShapeDtypeStruct((N_EDGES // 2, HIDDEN), jnp.float32),
        ],
        mesh=_sc_mesh(),
        compiler_params=_SC_PARAMS,
        scratch_types=[
            pltpu.VMEM_SHARED((N_NODES, HIDDEN), jnp.float32),
            pltpu.VMEM((2, K * ROW, HIDDEN), jnp.float32),
            pltpu.VMEM((2, K, ROW), jnp.int32),
            pltpu.SemaphoreType.DMA((2,)),
            pltpu.SemaphoreType.DMA((2,)),
        ],
    )
    return f(nt, src2, dst2)
